# baseline (speedup 1.0000x reference)
import functools

import jax
import jax.numpy as jnp
from jax import lax
from jax.experimental import pallas as pl
from jax.experimental.pallas import tpu as pltpu
from jax.experimental.pallas import tpu_sc as plsc
from jax.sharding import PartitionSpec as P

DIM = 32
K = 8192
N = 8192
TOKEN_BLOCK = 1024
_CHUNK_K = 2048


def _dist_argmin_body(x_ref, embt_ref, codes_ref):
    xb = x_ref[...]
    embt = embt_ref[...]
    a_sq = jnp.sum(xb * xb, axis=-1, keepdims=True)
    b_sq = jnp.sum(embt * embt, axis=0, keepdims=True)
    prod2 = lax.dot_general(
        -2.0 * xb, embt, (((1,), (0,)), ((), ())),
        preferred_element_type=jnp.float32,
    )
    iota_f = lax.broadcasted_iota(
        jnp.int32, (xb.shape[0], _CHUNK_K), 1).astype(jnp.float32)
    best_v = None
    best_i = None
    for c in range(K // _CHUNK_K):
        dslice = (a_sq + b_sq[:, c * _CHUNK_K:(c + 1) * _CHUNK_K]) + \
            prod2[:, c * _CHUNK_K:(c + 1) * _CHUNK_K]
        cmin = jnp.min(dslice, axis=1, keepdims=True)
        cidx = jnp.min(jnp.where(dslice == cmin, iota_f, float(_CHUNK_K)),
                       axis=1, keepdims=True).astype(jnp.int32) + c * _CHUNK_K
        if c == 0:
            best_v, best_i = cmin, cidx
        else:
            keep = (best_v < cmin) | ((best_v == cmin) & (best_i < cidx))
            best_v = jnp.where(keep, best_v, cmin)
            best_i = jnp.where(keep, best_i, cidx)
        best_v = best_v.astype(jnp.bfloat16).astype(jnp.float32)
    codes_ref[...] = best_i


def _dist_argmin(xf, embt, n_local):
    return pl.pallas_call(
        _dist_argmin_body,
        grid=(n_local // TOKEN_BLOCK,),
        in_specs=[
            pl.BlockSpec((TOKEN_BLOCK, DIM), lambda i: (i, 0)),
            pl.BlockSpec((DIM, K), lambda i: (0, 0)),
        ],
        out_specs=pl.BlockSpec((TOKEN_BLOCK, 1), lambda i: (i, 0)),
        out_shape=jax.ShapeDtypeStruct((n_local, 1), jnp.int32),
    )(xf, embt)


_NC, _NS = 2, 16
_NW = _NC * _NS
_CHUNK = 128


@functools.lru_cache(maxsize=8)
def _make_sc_gather(n_local):
    bpw = n_local // _NW
    nch = bpw // _CHUNK
    mesh = plsc.VectorSubcoreMesh(core_axis_name="c", subcore_axis_name="s")

    @functools.partial(
        pl.kernel,
        mesh=mesh,
        out_type=jax.ShapeDtypeStruct((n_local, DIM), jnp.float32),
        scratch_types=[
            pltpu.VMEM((nch, _CHUNK), jnp.int32),
            pltpu.VMEM((nch, _CHUNK, DIM), jnp.float32),
            pltpu.SemaphoreType.DMA,
        ],
        compiler_params=pltpu.CompilerParams(use_tc_tiling_on_sc=False),
    )
    def _sc_gather(table_hbm, idx_hbm, out_hbm, idx_v, rows_v, sem):
        wid = lax.axis_index("s") * _NC + lax.axis_index("c")
        pltpu.sync_copy(idx_hbm.at[pl.ds(wid * nch, nch)], idx_v)
        copies = [
            pltpu.async_copy(table_hbm.at[idx_v.at[j]], rows_v.at[j], sem)
            for j in range(nch)
        ]
        for c in copies:
            c.wait()
        for j in range(nch):
            pltpu.sync_copy(
                rows_v.at[j], out_hbm.at[pl.ds(wid * bpw + j * _CHUNK, _CHUNK)]
            )

    return _sc_gather


def _run_shard(xf, embedding):
    n_local = xf.shape[0]
    embt = embedding.T
    codes = jnp.reshape(_dist_argmin(xf, embt, n_local), (-1,))
    idx2d = jnp.reshape(codes, (-1, _CHUNK))
    quantized = _make_sc_gather(n_local)(embedding, idx2d)
    return quantized, xf, codes


def kernel(x, embedding):
    xf = jnp.reshape(x, (-1, x.shape[-1]))
    ndev = 2 if jax.device_count() >= 2 else 1
    if ndev == 1:
        return _run_shard(xf, embedding)
    mesh = jax.make_mesh((ndev,), ("d",))
    xf = jax.reshard(xf, jax.sharding.NamedSharding(mesh, P("d")))
    embedding = jax.reshard(
        embedding, jax.sharding.NamedSharding(mesh, P()))
    sharded = jax.shard_map(
        _run_shard, mesh=mesh,
        in_specs=(P("d"), P()),
        out_specs=(P("d"), P("d"), P("d")),
        check_vma=False,
    )
    return sharded(xf, embedding)


# speedup vs baseline: 4.2697x; 4.2697x over previous
import functools

import jax
import jax.numpy as jnp
from jax import lax
from jax.experimental import pallas as pl
from jax.experimental.pallas import tpu as pltpu
from jax.experimental.pallas import tpu_sc as plsc

DIM = 32
K = 8192
N = 8192
TOKEN_BLOCK = 1024
_CHUNK_K = 2048


def _dist_argmin_body(x_ref, embt_ref, codes_ref):
    xb = x_ref[...]
    embt = embt_ref[...]
    a_sq = jnp.sum(xb * xb, axis=-1, keepdims=True)
    b_sq = jnp.sum(embt * embt, axis=0, keepdims=True)
    prod2 = lax.dot_general(
        -2.0 * xb, embt, (((1,), (0,)), ((), ())),
        preferred_element_type=jnp.float32,
    )
    iota_f = lax.broadcasted_iota(
        jnp.int32, (xb.shape[0], _CHUNK_K), 1).astype(jnp.float32)
    best_v = None
    best_i = None
    for c in range(K // _CHUNK_K):
        dslice = (a_sq + b_sq[:, c * _CHUNK_K:(c + 1) * _CHUNK_K]) + \
            prod2[:, c * _CHUNK_K:(c + 1) * _CHUNK_K]
        cmin = jnp.min(dslice, axis=1, keepdims=True)
        cidx = jnp.min(jnp.where(dslice == cmin, iota_f, float(_CHUNK_K)),
                       axis=1, keepdims=True).astype(jnp.int32) + c * _CHUNK_K
        if c == 0:
            best_v, best_i = cmin, cidx
        else:
            keep = (best_v < cmin) | ((best_v == cmin) & (best_i < cidx))
            best_v = jnp.where(keep, best_v, cmin)
            best_i = jnp.where(keep, best_i, cidx)
        best_v = best_v.astype(jnp.bfloat16).astype(jnp.float32)
    codes_ref[...] = best_i


def _dist_argmin(xf, embt, n_local):
    return pl.pallas_call(
        _dist_argmin_body,
        grid=(n_local // TOKEN_BLOCK,),
        in_specs=[
            pl.BlockSpec((TOKEN_BLOCK, DIM), lambda i: (i, 0)),
            pl.BlockSpec((DIM, K), lambda i: (0, 0)),
        ],
        out_specs=pl.BlockSpec((TOKEN_BLOCK, 1), lambda i: (i, 0)),
        out_shape=jax.ShapeDtypeStruct((n_local, 1), jnp.int32),
    )(xf, embt)


_NC, _NS = 2, 16
_NW = _NC * _NS
_CHUNK = 128


@functools.lru_cache(maxsize=8)
def _make_sc_gather(n_local):
    bpw = n_local // _NW
    nch = bpw // _CHUNK
    mesh = plsc.VectorSubcoreMesh(core_axis_name="c", subcore_axis_name="s")

    @functools.partial(
        pl.kernel,
        mesh=mesh,
        out_type=jax.ShapeDtypeStruct((n_local, DIM), jnp.float32),
        scratch_types=[
            pltpu.VMEM((nch, _CHUNK), jnp.int32),
            pltpu.VMEM((nch, _CHUNK, DIM), jnp.float32),
            pltpu.SemaphoreType.DMA,
        ],
        compiler_params=pltpu.CompilerParams(use_tc_tiling_on_sc=False),
    )
    def _sc_gather(table_hbm, idx_hbm, out_hbm, idx_v, rows_v, sem):
        wid = lax.axis_index("s") * _NC + lax.axis_index("c")
        pltpu.sync_copy(idx_hbm.at[pl.ds(wid * nch, nch)], idx_v)
        copies = [
            pltpu.async_copy(table_hbm.at[idx_v.at[j]], rows_v.at[j], sem)
            for j in range(nch)
        ]
        for c in copies:
            c.wait()
        for j in range(nch):
            pltpu.sync_copy(
                rows_v.at[j], out_hbm.at[pl.ds(wid * bpw + j * _CHUNK, _CHUNK)]
            )

    return _sc_gather


def _run_shard(xf, embedding):
    n_local = xf.shape[0]
    embt = embedding.T
    codes = jnp.reshape(_dist_argmin(xf, embt, n_local), (-1,))
    idx2d = jnp.reshape(codes, (-1, _CHUNK))
    quantized = _make_sc_gather(n_local)(embedding, idx2d)
    return quantized, xf, codes


def kernel(x, embedding):
    xf = jnp.reshape(x, (-1, x.shape[-1]))
    return _run_shard(xf, embedding)
